# baseline (speedup 1.0000x reference)
import jax
import jax.numpy as jnp
from jax import lax
from jax.experimental import pallas as pl
from jax.experimental.pallas import tpu as pltpu

H, A = 100, 32
HA = H * A
POP = 2048
NUM_TOPK = 205
NUM_ITERS = 5
MOMENTUM = 0.1

C = 256
NB = POP // C
LG = HA // 128


def _cumsum_lanes(x):
    n = x.shape[1]
    k = 1
    while k < n:
        shifted = jnp.pad(x, ((0, 0), (k, 0)))[:, :n]
        x = x + shifted
        k *= 2
    return x


def _cem_kernel(eps_a, eps_b, means0_ref, tgt_ref, q4_ref, out_ref,
                vals, w, bh, loc, scale, bva, besta,
                acc_e, acc_e2, acc_b):
    i = pl.program_id(0)
    j = pl.program_id(1)

    @pl.when((i == 0) & (j == 0))
    def _init():
        loc[...] = means0_ref[...]
        scale[...] = jnp.ones_like(scale)
        bva[0] = -jnp.inf
        besta[...] = jnp.zeros_like(besta)

    @pl.when(j < NB)
    def _values():
        eps = eps_a[0]
        d = (loc[...] + scale[...] * eps) - tgt_ref[...]
        q4 = q4_ref[...]
        acc = jnp.zeros((C, 128), jnp.float32)
        for k in range(LG):
            dk = d[:, 128 * k:128 * (k + 1)]
            ek = lax.dot_general(dk, q4, (((1,), (0,)), ((), ())),
                                 preferred_element_type=jnp.float32)
            acc = acc + ek * dk
        vblock = -jnp.sum(acc, axis=1, keepdims=True).reshape(1, C)
        vals[pl.ds(0, 1), pl.ds(j * C, C)] = vblock

    @pl.when(j == NB)
    def _select():
        v = vals[...]
        b = v.view(jnp.int32)
        key = jnp.where(b < 0, b ^ jnp.int32(0x7FFFFFFF), b)
        ukey = key.view(jnp.uint32) ^ jnp.uint32(0x80000000)

        def body(k, t):
            t_try = t | (jnp.uint32(1) << jnp.uint32(31 - k))
            cnt = jnp.sum((ukey >= t_try).astype(jnp.int32))
            return jnp.where(cnt >= NUM_TOPK, t_try, t)

        t = lax.fori_loop(0, 32, body, jnp.uint32(0))
        gt = ukey > t
        eq = ukey == t
        need = NUM_TOPK - jnp.sum(gt.astype(jnp.int32))
        rank = _cumsum_lanes(eq.astype(jnp.float32))
        wsel = gt | (eq & (rank <= need.astype(jnp.float32)))
        w[...] = wsel.astype(jnp.float32)
        kmax = jnp.max(key)
        eqb = key == kmax
        bh[...] = (eqb & (_cumsum_lanes(eqb.astype(jnp.float32)) == 1.0)
                   ).astype(jnp.float32)
        bva[1] = jnp.max(v)
        acc_e[...] = jnp.zeros_like(acc_e)
        acc_e2[...] = jnp.zeros_like(acc_e2)
        acc_b[...] = jnp.zeros_like(acc_b)

    @pl.when(j > NB)
    def _accumulate():
        k = j - NB - 1
        e = eps_b[0]
        wk = w[pl.ds(0, 1), pl.ds(k * C, C)]
        bhk = bh[pl.ds(0, 1), pl.ds(k * C, C)]
        dn = (((1,), (0,)), ((), ()))
        hp = lax.Precision.HIGHEST
        acc_e[...] += lax.dot_general(wk, e, dn, precision=hp,
                                      preferred_element_type=jnp.float32)
        acc_e2[...] += lax.dot_general(wk, e * e, dn, precision=hp,
                                       preferred_element_type=jnp.float32)
        acc_b[...] += lax.dot_general(bhk, e, dn, precision=hp,
                                      preferred_element_type=jnp.float32)

    @pl.when(j == 2 * NB)
    def _finalize():
        lc = loc[...]
        sc = scale[...]
        inv = jnp.float32(1.0 / NUM_TOPK)
        m_e = acc_e[...] * inv
        m_e2 = acc_e2[...] * inv
        new_means = lc + sc * m_e
        var_eps = (m_e2 - m_e * m_e) * jnp.float32(NUM_TOPK / (NUM_TOPK - 1))
        new_stds = sc * jnp.sqrt(jnp.maximum(var_eps, 0.0))
        best_sample = lc + sc * acc_b[...]
        bvi = bva[1]
        better = bvi > bva[0]
        bva[0] = jnp.where(better, bvi, bva[0])
        besta[...] = jnp.where(better, best_sample, besta[...])
        loc[...] = (jnp.float32(MOMENTUM) * means0_ref[...]
                    + jnp.float32(1.0 - MOMENTUM) * new_means)
        scale[...] = (jnp.float32(MOMENTUM)
                      + jnp.float32(1.0 - MOMENTUM) * new_stds)

    @pl.when((i == NUM_ITERS - 1) & (j == 2 * NB))
    def _emit():
        out_ref[...] = besta[...]


def kernel(initial_solution, target, Q):
    means0 = initial_solution.reshape(1, HA)
    tgt = target.reshape(1, HA)
    q4 = jnp.kron(jnp.eye(4, dtype=jnp.float32), Q)

    base = jax.random.key(42)
    eps_all = jnp.stack([
        jax.random.normal(jax.random.fold_in(base, i),
                          (POP, H, A), jnp.float32).reshape(POP, HA)
        for i in range(NUM_ITERS)])

    out = pl.pallas_call(
        _cem_kernel,
        grid=(NUM_ITERS, 2 * NB + 1),
        in_specs=[
            pl.BlockSpec((1, C, HA),
                         lambda i, j: (i, jnp.minimum(j, NB - 1), 0)),
            pl.BlockSpec((1, C, HA),
                         lambda i, j: (i, jnp.clip(j - NB - 1, 0, NB - 1), 0)),
            pl.BlockSpec((1, HA), lambda i, j: (0, 0)),
            pl.BlockSpec((1, HA), lambda i, j: (0, 0)),
            pl.BlockSpec((128, 128), lambda i, j: (0, 0)),
        ],
        out_specs=pl.BlockSpec((1, HA), lambda i, j: (0, 0)),
        out_shape=jax.ShapeDtypeStruct((1, HA), jnp.float32),
        scratch_shapes=[
            pltpu.VMEM((1, POP), jnp.float32),
            pltpu.VMEM((1, POP), jnp.float32),
            pltpu.VMEM((1, POP), jnp.float32),
            pltpu.VMEM((1, HA), jnp.float32),
            pltpu.VMEM((1, HA), jnp.float32),
            pltpu.SMEM((2,), jnp.float32),
            pltpu.VMEM((1, HA), jnp.float32),
            pltpu.VMEM((1, HA), jnp.float32),
            pltpu.VMEM((1, HA), jnp.float32),
            pltpu.VMEM((1, HA), jnp.float32),
        ],
        compiler_params=pltpu.CompilerParams(
            dimension_semantics=("arbitrary", "arbitrary")),
    )(eps_all, eps_all, means0, tgt, q4)
    return out.reshape(H, A)


# speedup vs baseline: 1.3265x; 1.3265x over previous
import jax
import jax.numpy as jnp
from jax import lax
from jax.experimental import pallas as pl
from jax.experimental.pallas import tpu as pltpu

H, A = 100, 32
HA = H * A
POP = 2048
NUM_TOPK = 205
NUM_ITERS = 5
MOMENTUM = 0.1

C = 256
NB = POP // C
LG = HA // 128


def _cumsum_lanes(x):
    n = x.shape[1]
    k = 1
    while k < n:
        shifted = jnp.pad(x, ((0, 0), (k, 0)))[:, :n]
        x = x + shifted
        k *= 2
    return x


def _cem_kernel(eps_a, eps_b, means0_ref, tgt_ref, q4_ref, out_ref,
                vals, w, bh, loc, scale, bva, besta,
                acc_e, acc_e2, acc_b):
    i = pl.program_id(0)
    j = pl.program_id(1)

    @pl.when((i == 0) & (j == 0))
    def _init():
        loc[...] = means0_ref[...]
        scale[...] = jnp.ones_like(scale)
        bva[0] = -jnp.inf
        besta[...] = jnp.zeros_like(besta)

    @pl.when(j < NB)
    def _values():
        eps = eps_a[0]
        d = (loc[...] + scale[...] * eps) - tgt_ref[...]
        q4 = q4_ref[...]
        acc = jnp.zeros((C, 128), jnp.float32)
        for k in range(LG):
            dk = d[:, 128 * k:128 * (k + 1)]
            ek = lax.dot_general(dk, q4, (((1,), (0,)), ((), ())),
                                 preferred_element_type=jnp.float32)
            acc = acc + ek * dk
        vblock = -jnp.sum(acc, axis=1, keepdims=True).reshape(1, C)
        vals[pl.ds(0, 1), pl.ds(j * C, C)] = vblock

    @pl.when(j == NB)
    def _select():
        v = vals[...]
        b = v.view(jnp.int32)
        key = jnp.where(b < 0, b ^ jnp.int32(0x7FFFFFFF), b)
        ukey = key.view(jnp.uint32) ^ jnp.uint32(0x80000000)

        def body(k, t):
            t_try = t | (jnp.uint32(1) << jnp.uint32(31 - k))
            cnt = jnp.sum((ukey >= t_try).astype(jnp.int32))
            return jnp.where(cnt >= NUM_TOPK, t_try, t)

        t = lax.fori_loop(0, 32, body, jnp.uint32(0))
        gt = ukey > t
        eq = ukey == t
        need = NUM_TOPK - jnp.sum(gt.astype(jnp.int32))
        rank = _cumsum_lanes(eq.astype(jnp.float32))
        wsel = gt | (eq & (rank <= need.astype(jnp.float32)))
        w[...] = wsel.astype(jnp.float32)
        kmax = jnp.max(key)
        eqb = key == kmax
        bh[...] = (eqb & (_cumsum_lanes(eqb.astype(jnp.float32)) == 1.0)
                   ).astype(jnp.float32)
        bva[1] = jnp.max(v)
        acc_e[...] = jnp.zeros_like(acc_e)
        acc_e2[...] = jnp.zeros_like(acc_e2)
        acc_b[...] = jnp.zeros_like(acc_b)

    @pl.when(j > NB)
    def _accumulate():
        k = j - NB - 1
        e = eps_b[0]
        wk = w[pl.ds(0, 1), pl.ds(k * C, C)]
        bhk = bh[pl.ds(0, 1), pl.ds(k * C, C)]
        dn = (((1,), (0,)), ((), ()))
        acc_e[...] += lax.dot_general(wk, e, dn,
                                      preferred_element_type=jnp.float32)
        acc_e2[...] += lax.dot_general(wk, e * e, dn,
                                       preferred_element_type=jnp.float32)
        acc_b[...] += lax.dot_general(bhk, e, dn,
                                      preferred_element_type=jnp.float32)

    @pl.when(j == 2 * NB)
    def _finalize():
        lc = loc[...]
        sc = scale[...]
        inv = jnp.float32(1.0 / NUM_TOPK)
        m_e = acc_e[...] * inv
        m_e2 = acc_e2[...] * inv
        new_means = lc + sc * m_e
        var_eps = (m_e2 - m_e * m_e) * jnp.float32(NUM_TOPK / (NUM_TOPK - 1))
        new_stds = sc * jnp.sqrt(jnp.maximum(var_eps, 0.0))
        best_sample = lc + sc * acc_b[...]
        bvi = bva[1]
        better = bvi > bva[0]
        bva[0] = jnp.where(better, bvi, bva[0])
        besta[...] = jnp.where(better, best_sample, besta[...])
        loc[...] = (jnp.float32(MOMENTUM) * means0_ref[...]
                    + jnp.float32(1.0 - MOMENTUM) * new_means)
        scale[...] = (jnp.float32(MOMENTUM)
                      + jnp.float32(1.0 - MOMENTUM) * new_stds)

    @pl.when((i == NUM_ITERS - 1) & (j == 2 * NB))
    def _emit():
        out_ref[...] = besta[...]


def kernel(initial_solution, target, Q):
    means0 = initial_solution.reshape(1, HA)
    tgt = target.reshape(1, HA)
    q4 = jnp.kron(jnp.eye(4, dtype=jnp.float32), Q)

    base = jax.random.key(42)
    keys = jnp.stack([jax.random.fold_in(base, i) for i in range(NUM_ITERS)])
    eps_all = jax.vmap(
        lambda k: jax.random.normal(k, (POP, HA), jnp.float32))(keys)

    out = pl.pallas_call(
        _cem_kernel,
        grid=(NUM_ITERS, 2 * NB + 1),
        in_specs=[
            pl.BlockSpec((1, C, HA),
                         lambda i, j: (i, jnp.minimum(j, NB - 1), 0)),
            pl.BlockSpec((1, C, HA),
                         lambda i, j: (i, jnp.clip(j - NB - 1, 0, NB - 1), 0)),
            pl.BlockSpec((1, HA), lambda i, j: (0, 0)),
            pl.BlockSpec((1, HA), lambda i, j: (0, 0)),
            pl.BlockSpec((128, 128), lambda i, j: (0, 0)),
        ],
        out_specs=pl.BlockSpec((1, HA), lambda i, j: (0, 0)),
        out_shape=jax.ShapeDtypeStruct((1, HA), jnp.float32),
        scratch_shapes=[
            pltpu.VMEM((1, POP), jnp.float32),
            pltpu.VMEM((1, POP), jnp.float32),
            pltpu.VMEM((1, POP), jnp.float32),
            pltpu.VMEM((1, HA), jnp.float32),
            pltpu.VMEM((1, HA), jnp.float32),
            pltpu.SMEM((2,), jnp.float32),
            pltpu.VMEM((1, HA), jnp.float32),
            pltpu.VMEM((1, HA), jnp.float32),
            pltpu.VMEM((1, HA), jnp.float32),
            pltpu.VMEM((1, HA), jnp.float32),
        ],
        compiler_params=pltpu.CompilerParams(
            dimension_semantics=("arbitrary", "arbitrary")),
    )(eps_all, eps_all, means0, tgt, q4)
    return out.reshape(H, A)


# speedup vs baseline: 1.4048x; 1.0591x over previous
import math

import jax
import jax.numpy as jnp
from jax import lax
from jax.experimental import pallas as pl
from jax.experimental.pallas import tpu as pltpu

H, A = 100, 32
HA = H * A
POP = 2048
NUM_TOPK = 205
NUM_ITERS = 5
MOMENTUM = 0.1

C = 256
NB = POP // C
LG = HA // 128
GEN = 16
HALF = POP * HA // 2


def _cumsum_lanes(x):
    n = x.shape[1]
    k = 1
    while k < n:
        shifted = jnp.pad(x, ((0, 0), (k, 0)))[:, :n]
        x = x + shifted
        k *= 2
    return x


def _erfinv(x):
    w = -jnp.log((1.0 - x) * (1.0 + x))
    ws = w - 2.5
    p1 = jnp.float32(2.81022636e-08)
    for c in (3.43273939e-07, -3.5233877e-06, -4.39150654e-06,
              0.00021858087, -0.00125372503, -0.00417768164,
              0.246640727, 1.50140941):
        p1 = p1 * ws + jnp.float32(c)
    wl = jnp.sqrt(w) - 3.0
    p2 = jnp.float32(-0.000200214257)
    for c in (0.000100950558, 0.00134934322, -0.00367342844,
              0.00573950773, -0.0076224613, 0.00943887047,
              1.00167406, 2.83297682):
        p2 = p2 * wl + jnp.float32(c)
    return jnp.where(w < 5.0, p1, p2) * x


def _cem_kernel(keys_ref, means0_ref, tgt_ref, q4_ref, out_ref,
                vals, w, bh, loc, scale, bva, besta,
                acc_e, acc_e2, acc_b, eps_scr):
    i = pl.program_id(0)
    j = pl.program_id(1)

    @pl.when((i == 0) & (j == 0))
    def _init():
        loc[...] = means0_ref[...]
        scale[...] = jnp.ones_like(scale)
        bva[0] = -jnp.inf
        besta[...] = jnp.zeros_like(besta)

    @pl.when(j < NB)
    def _generate_and_values():
        k0 = keys_ref[i, 0]
        k1 = keys_ref[i, 1]
        k2 = k0 ^ k1 ^ jnp.uint32(0x1BD11BDA)
        ks = (k0, k1, k2)

        def gen_body(sc, carry):
            r0 = j * C + sc * GEN
            p = ((r0 * HA).astype(jnp.uint32)
                 + lax.broadcasted_iota(jnp.uint32, (GEN, HA), 0)
                 * jnp.uint32(HA)
                 + lax.broadcasted_iota(jnp.uint32, (GEN, HA), 1))
            x0 = jnp.zeros_like(p) + k0
            x1 = p + k1
            rots = ((13, 15, 26, 6), (17, 29, 16, 24))
            for g in range(5):
                for r in rots[g % 2]:
                    x0 = x0 + x1
                    x1 = (x1 << jnp.uint32(r)) | (x1 >> jnp.uint32(32 - r))
                    x1 = x1 ^ x0
                x0 = x0 + ks[(g + 1) % 3]
                x1 = x1 + ks[(g + 2) % 3] + jnp.uint32(g + 1)
            bits = x0 ^ x1
            fl = lax.bitcast_convert_type(
                (bits >> jnp.uint32(9)) | jnp.uint32(0x3F800000), jnp.float32)
            lo = jnp.float32(-0.99999994)
            u = (fl - 1.0) * (jnp.float32(1.0) - lo) + lo
            u = jnp.maximum(lo, u)
            eps_scr[pl.ds(r0, GEN), :] = (
                jnp.float32(math.sqrt(2)) * _erfinv(u))
            return carry

        lax.fori_loop(0, C // GEN, gen_body, 0)

        eps = eps_scr[pl.ds(j * C, C), :]
        d = (loc[...] + scale[...] * eps) - tgt_ref[...]
        q4 = q4_ref[...]
        acc = jnp.zeros((C, 128), jnp.float32)
        for k in range(LG):
            dk = d[:, 128 * k:128 * (k + 1)]
            ek = lax.dot_general(dk, q4, (((1,), (0,)), ((), ())),
                                 preferred_element_type=jnp.float32)
            acc = acc + ek * dk
        vblock = -jnp.sum(acc, axis=1, keepdims=True).reshape(1, C)
        vals[pl.ds(0, 1), pl.ds(j * C, C)] = vblock

    @pl.when(j == NB)
    def _select():
        v = vals[...]
        b = v.view(jnp.int32)
        key = jnp.where(b < 0, b ^ jnp.int32(0x7FFFFFFF), b)
        ukey = key.view(jnp.uint32) ^ jnp.uint32(0x80000000)

        def body(k, t):
            t_try = t | (jnp.uint32(1) << jnp.uint32(31 - k))
            cnt = jnp.sum((ukey >= t_try).astype(jnp.int32))
            return jnp.where(cnt >= NUM_TOPK, t_try, t)

        t = lax.fori_loop(0, 32, body, jnp.uint32(0))
        gt = ukey > t
        eq = ukey == t
        need = NUM_TOPK - jnp.sum(gt.astype(jnp.int32))
        rank = _cumsum_lanes(eq.astype(jnp.float32))
        wsel = gt | (eq & (rank <= need.astype(jnp.float32)))
        w[...] = wsel.astype(jnp.float32)
        kmax = jnp.max(key)
        eqb = key == kmax
        bh[...] = (eqb & (_cumsum_lanes(eqb.astype(jnp.float32)) == 1.0)
                   ).astype(jnp.float32)
        bva[1] = jnp.max(v)
        acc_e[...] = jnp.zeros_like(acc_e)
        acc_e2[...] = jnp.zeros_like(acc_e2)
        acc_b[...] = jnp.zeros_like(acc_b)

    @pl.when(j > NB)
    def _accumulate():
        k = j - NB - 1
        e = eps_scr[pl.ds(k * C, C), :]
        wk = w[pl.ds(0, 1), pl.ds(k * C, C)]
        bhk = bh[pl.ds(0, 1), pl.ds(k * C, C)]
        dn = (((1,), (0,)), ((), ()))
        acc_e[...] += lax.dot_general(wk, e, dn,
                                      preferred_element_type=jnp.float32)
        acc_e2[...] += lax.dot_general(wk, e * e, dn,
                                       preferred_element_type=jnp.float32)
        acc_b[...] += lax.dot_general(bhk, e, dn,
                                      preferred_element_type=jnp.float32)

    @pl.when(j == 2 * NB)
    def _finalize():
        lc = loc[...]
        sc = scale[...]
        inv = jnp.float32(1.0 / NUM_TOPK)
        m_e = acc_e[...] * inv
        m_e2 = acc_e2[...] * inv
        new_means = lc + sc * m_e
        var_eps = (m_e2 - m_e * m_e) * jnp.float32(NUM_TOPK / (NUM_TOPK - 1))
        new_stds = sc * jnp.sqrt(jnp.maximum(var_eps, 0.0))
        best_sample = lc + sc * acc_b[...]
        bvi = bva[1]
        better = bvi > bva[0]
        bva[0] = jnp.where(better, bvi, bva[0])
        besta[...] = jnp.where(better, best_sample, besta[...])
        loc[...] = (jnp.float32(MOMENTUM) * means0_ref[...]
                    + jnp.float32(1.0 - MOMENTUM) * new_means)
        scale[...] = (jnp.float32(MOMENTUM)
                      + jnp.float32(1.0 - MOMENTUM) * new_stds)

    @pl.when((i == NUM_ITERS - 1) & (j == 2 * NB))
    def _emit():
        out_ref[...] = besta[...]


def kernel(initial_solution, target, Q):
    means0 = initial_solution.reshape(1, HA)
    tgt = target.reshape(1, HA)
    q4 = jnp.kron(jnp.eye(4, dtype=jnp.float32), Q)

    base = jax.random.key(42)
    keys = jnp.stack([jax.random.key_data(jax.random.fold_in(base, i))
                      for i in range(NUM_ITERS)]).astype(jnp.uint32)

    out = pl.pallas_call(
        _cem_kernel,
        grid=(NUM_ITERS, 2 * NB + 1),
        in_specs=[
            pl.BlockSpec(memory_space=pltpu.SMEM),
            pl.BlockSpec((1, HA), lambda i, j: (0, 0)),
            pl.BlockSpec((1, HA), lambda i, j: (0, 0)),
            pl.BlockSpec((128, 128), lambda i, j: (0, 0)),
        ],
        out_specs=pl.BlockSpec((1, HA), lambda i, j: (0, 0)),
        out_shape=jax.ShapeDtypeStruct((1, HA), jnp.float32),
        scratch_shapes=[
            pltpu.VMEM((1, POP), jnp.float32),
            pltpu.VMEM((1, POP), jnp.float32),
            pltpu.VMEM((1, POP), jnp.float32),
            pltpu.VMEM((1, HA), jnp.float32),
            pltpu.VMEM((1, HA), jnp.float32),
            pltpu.SMEM((2,), jnp.float32),
            pltpu.VMEM((1, HA), jnp.float32),
            pltpu.VMEM((1, HA), jnp.float32),
            pltpu.VMEM((1, HA), jnp.float32),
            pltpu.VMEM((1, HA), jnp.float32),
            pltpu.VMEM((POP, HA), jnp.float32),
        ],
        compiler_params=pltpu.CompilerParams(
            dimension_semantics=("arbitrary", "arbitrary")),
    )(keys, means0, tgt, q4)
    return out.reshape(H, A)


# speedup vs baseline: 1.4365x; 1.0225x over previous
import math

import jax
import jax.numpy as jnp
from jax import lax
from jax.experimental import pallas as pl
from jax.experimental.pallas import tpu as pltpu

H, A = 100, 32
HA = H * A
POP = 2048
NUM_TOPK = 205
NUM_ITERS = 5
MOMENTUM = 0.1

C = 256
NB = POP // C
LG = HA // 128
GEN = 32
HALF = POP * HA // 2


def _cumsum_lanes(x):
    n = x.shape[1]
    k = 1
    while k < n:
        shifted = jnp.pad(x, ((0, 0), (k, 0)))[:, :n]
        x = x + shifted
        k *= 2
    return x


def _erfinv(x):
    w = -jnp.log((1.0 - x) * (1.0 + x))
    ws = w - 2.5
    p1 = jnp.float32(2.81022636e-08)
    for c in (3.43273939e-07, -3.5233877e-06, -4.39150654e-06,
              0.00021858087, -0.00125372503, -0.00417768164,
              0.246640727, 1.50140941):
        p1 = p1 * ws + jnp.float32(c)
    wl = jnp.sqrt(w) - 3.0
    p2 = jnp.float32(-0.000200214257)
    for c in (0.000100950558, 0.00134934322, -0.00367342844,
              0.00573950773, -0.0076224613, 0.00943887047,
              1.00167406, 2.83297682):
        p2 = p2 * wl + jnp.float32(c)
    return jnp.where(w < 5.0, p1, p2) * x


def _cem_kernel(keys_ref, means0_ref, tgt_ref, q4_ref, out_ref,
                vals, w, bh, loc, scale, bva, besta,
                acc_e, acc_e2, acc_b, eps_scr):
    i = pl.program_id(0)
    j = pl.program_id(1)

    @pl.when((i == 0) & (j == 0))
    def _init():
        loc[...] = means0_ref[...]
        scale[...] = jnp.ones_like(scale)
        bva[0] = -jnp.inf
        besta[...] = jnp.zeros_like(besta)

    @pl.when(j < NB)
    def _generate_and_values():
        k0 = keys_ref[i, 0]
        k1 = keys_ref[i, 1]
        k2 = k0 ^ k1 ^ jnp.uint32(0x1BD11BDA)
        ks = (k0, k1, k2)

        def gen_body(sc, carry):
            r0 = j * C + sc * GEN
            p = ((r0 * HA).astype(jnp.uint32)
                 + lax.broadcasted_iota(jnp.uint32, (GEN, HA), 0)
                 * jnp.uint32(HA)
                 + lax.broadcasted_iota(jnp.uint32, (GEN, HA), 1))
            x0 = jnp.zeros_like(p) + k0
            x1 = p + k1
            rots = ((13, 15, 26, 6), (17, 29, 16, 24))
            for g in range(5):
                for r in rots[g % 2]:
                    x0 = x0 + x1
                    x1 = (x1 << jnp.uint32(r)) | (x1 >> jnp.uint32(32 - r))
                    x1 = x1 ^ x0
                x0 = x0 + ks[(g + 1) % 3]
                x1 = x1 + ks[(g + 2) % 3] + jnp.uint32(g + 1)
            bits = x0 ^ x1
            fl = lax.bitcast_convert_type(
                (bits >> jnp.uint32(9)) | jnp.uint32(0x3F800000), jnp.float32)
            lo = jnp.float32(-0.99999994)
            u = (fl - 1.0) * (jnp.float32(1.0) - lo) + lo
            u = jnp.maximum(lo, u)
            eps_scr[pl.ds(r0, GEN), :] = (
                jnp.float32(math.sqrt(2)) * _erfinv(u))
            return carry

        lax.fori_loop(0, C // GEN, gen_body, 0)

        eps = eps_scr[pl.ds(j * C, C), :]
        d = (loc[...] + scale[...] * eps) - tgt_ref[...]
        q4 = q4_ref[...]
        acc = jnp.zeros((C, 128), jnp.float32)
        for k in range(LG):
            dk = d[:, 128 * k:128 * (k + 1)]
            ek = lax.dot_general(dk, q4, (((1,), (0,)), ((), ())),
                                 preferred_element_type=jnp.float32)
            acc = acc + ek * dk
        vblock = -jnp.sum(acc, axis=1, keepdims=True).reshape(1, C)
        vals[pl.ds(0, 1), pl.ds(j * C, C)] = vblock

    @pl.when(j == NB)
    def _select():
        v = vals[...]
        b = v.view(jnp.int32)
        key = jnp.where(b < 0, b ^ jnp.int32(0x7FFFFFFF), b)
        ukey = key.view(jnp.uint32) ^ jnp.uint32(0x80000000)

        def body(k, t):
            t_try = t | (jnp.uint32(1) << jnp.uint32(31 - k))
            cnt = jnp.sum((ukey >= t_try).astype(jnp.int32))
            return jnp.where(cnt >= NUM_TOPK, t_try, t)

        t = lax.fori_loop(0, 32, body, jnp.uint32(0))
        gt = ukey > t
        eq = ukey == t
        need = NUM_TOPK - jnp.sum(gt.astype(jnp.int32))
        rank = _cumsum_lanes(eq.astype(jnp.float32))
        wsel = gt | (eq & (rank <= need.astype(jnp.float32)))
        w[...] = wsel.astype(jnp.float32)
        kmax = jnp.max(key)
        eqb = key == kmax
        bh[...] = (eqb & (_cumsum_lanes(eqb.astype(jnp.float32)) == 1.0)
                   ).astype(jnp.float32)
        bva[1] = jnp.max(v)
        acc_e[...] = jnp.zeros_like(acc_e)
        acc_e2[...] = jnp.zeros_like(acc_e2)
        acc_b[...] = jnp.zeros_like(acc_b)

    @pl.when(j > NB)
    def _accumulate():
        k = j - NB - 1
        e = eps_scr[pl.ds(k * C, C), :]
        wk = w[pl.ds(0, 1), pl.ds(k * C, C)]
        bhk = bh[pl.ds(0, 1), pl.ds(k * C, C)]
        dn = (((1,), (0,)), ((), ()))
        acc_e[...] += lax.dot_general(wk, e, dn,
                                      preferred_element_type=jnp.float32)
        acc_e2[...] += lax.dot_general(wk, e * e, dn,
                                       preferred_element_type=jnp.float32)
        acc_b[...] += lax.dot_general(bhk, e, dn,
                                      preferred_element_type=jnp.float32)

    @pl.when(j == 2 * NB)
    def _finalize():
        lc = loc[...]
        sc = scale[...]
        inv = jnp.float32(1.0 / NUM_TOPK)
        m_e = acc_e[...] * inv
        m_e2 = acc_e2[...] * inv
        new_means = lc + sc * m_e
        var_eps = (m_e2 - m_e * m_e) * jnp.float32(NUM_TOPK / (NUM_TOPK - 1))
        new_stds = sc * jnp.sqrt(jnp.maximum(var_eps, 0.0))
        best_sample = lc + sc * acc_b[...]
        bvi = bva[1]
        better = bvi > bva[0]
        bva[0] = jnp.where(better, bvi, bva[0])
        besta[...] = jnp.where(better, best_sample, besta[...])
        loc[...] = (jnp.float32(MOMENTUM) * means0_ref[...]
                    + jnp.float32(1.0 - MOMENTUM) * new_means)
        scale[...] = (jnp.float32(MOMENTUM)
                      + jnp.float32(1.0 - MOMENTUM) * new_stds)

    @pl.when((i == NUM_ITERS - 1) & (j == 2 * NB))
    def _emit():
        out_ref[...] = besta[...]


def kernel(initial_solution, target, Q):
    means0 = initial_solution.reshape(1, HA)
    tgt = target.reshape(1, HA)
    q4 = jnp.kron(jnp.eye(4, dtype=jnp.float32), Q)

    base = jax.random.key(42)
    keys = jnp.stack([jax.random.key_data(jax.random.fold_in(base, i))
                      for i in range(NUM_ITERS)]).astype(jnp.uint32)

    out = pl.pallas_call(
        _cem_kernel,
        grid=(NUM_ITERS, 2 * NB + 1),
        in_specs=[
            pl.BlockSpec(memory_space=pltpu.SMEM),
            pl.BlockSpec((1, HA), lambda i, j: (0, 0)),
            pl.BlockSpec((1, HA), lambda i, j: (0, 0)),
            pl.BlockSpec((128, 128), lambda i, j: (0, 0)),
        ],
        out_specs=pl.BlockSpec((1, HA), lambda i, j: (0, 0)),
        out_shape=jax.ShapeDtypeStruct((1, HA), jnp.float32),
        scratch_shapes=[
            pltpu.VMEM((1, POP), jnp.float32),
            pltpu.VMEM((1, POP), jnp.float32),
            pltpu.VMEM((1, POP), jnp.float32),
            pltpu.VMEM((1, HA), jnp.float32),
            pltpu.VMEM((1, HA), jnp.float32),
            pltpu.SMEM((2,), jnp.float32),
            pltpu.VMEM((1, HA), jnp.float32),
            pltpu.VMEM((1, HA), jnp.float32),
            pltpu.VMEM((1, HA), jnp.float32),
            pltpu.VMEM((1, HA), jnp.float32),
            pltpu.VMEM((POP, HA), jnp.float32),
        ],
        compiler_params=pltpu.CompilerParams(
            dimension_semantics=("arbitrary", "arbitrary")),
    )(keys, means0, tgt, q4)
    return out.reshape(H, A)


# speedup vs baseline: 1.4549x; 1.0128x over previous
import math

import jax
import jax.numpy as jnp
from jax import lax
from jax.experimental import pallas as pl
from jax.experimental.pallas import tpu as pltpu

H, A = 100, 32
HA = H * A
POP = 2048
NUM_TOPK = 205
NUM_ITERS = 5
MOMENTUM = 0.1

C = 256
NB = POP // C
LG = HA // 128
GEN = 64
HALF = POP * HA // 2


def _cumsum_lanes(x):
    n = x.shape[1]
    k = 1
    while k < n:
        shifted = jnp.pad(x, ((0, 0), (k, 0)))[:, :n]
        x = x + shifted
        k *= 2
    return x


def _erfinv(x):
    w = -jnp.log((1.0 - x) * (1.0 + x))
    ws = w - 2.5
    p1 = jnp.float32(2.81022636e-08)
    for c in (3.43273939e-07, -3.5233877e-06, -4.39150654e-06,
              0.00021858087, -0.00125372503, -0.00417768164,
              0.246640727, 1.50140941):
        p1 = p1 * ws + jnp.float32(c)
    wl = jnp.sqrt(w) - 3.0
    p2 = jnp.float32(-0.000200214257)
    for c in (0.000100950558, 0.00134934322, -0.00367342844,
              0.00573950773, -0.0076224613, 0.00943887047,
              1.00167406, 2.83297682):
        p2 = p2 * wl + jnp.float32(c)
    return jnp.where(w < 5.0, p1, p2) * x


def _cem_kernel(keys_ref, means0_ref, tgt_ref, q4_ref, out_ref,
                vals, w, bh, loc, scale, bva, besta,
                acc_e, acc_e2, acc_b, eps_scr):
    i = pl.program_id(0)
    j = pl.program_id(1)

    @pl.when((i == 0) & (j == 0))
    def _init():
        loc[...] = means0_ref[...]
        scale[...] = jnp.ones_like(scale)
        bva[0] = -jnp.inf
        besta[...] = jnp.zeros_like(besta)

    @pl.when(j < NB)
    def _generate_and_values():
        k0 = keys_ref[i, 0]
        k1 = keys_ref[i, 1]
        k2 = k0 ^ k1 ^ jnp.uint32(0x1BD11BDA)
        ks = (k0, k1, k2)

        def gen_body(sc, carry):
            r0 = j * C + sc * GEN
            p = ((r0 * HA).astype(jnp.uint32)
                 + lax.broadcasted_iota(jnp.uint32, (GEN, HA), 0)
                 * jnp.uint32(HA)
                 + lax.broadcasted_iota(jnp.uint32, (GEN, HA), 1))
            x0 = jnp.zeros_like(p) + k0
            x1 = p + k1
            rots = ((13, 15, 26, 6), (17, 29, 16, 24))
            for g in range(5):
                for r in rots[g % 2]:
                    x0 = x0 + x1
                    x1 = (x1 << jnp.uint32(r)) | (x1 >> jnp.uint32(32 - r))
                    x1 = x1 ^ x0
                x0 = x0 + ks[(g + 1) % 3]
                x1 = x1 + ks[(g + 2) % 3] + jnp.uint32(g + 1)
            bits = x0 ^ x1
            fl = lax.bitcast_convert_type(
                (bits >> jnp.uint32(9)) | jnp.uint32(0x3F800000), jnp.float32)
            lo = jnp.float32(-0.99999994)
            u = (fl - 1.0) * (jnp.float32(1.0) - lo) + lo
            u = jnp.maximum(lo, u)
            eps_scr[pl.ds(r0, GEN), :] = (
                jnp.float32(math.sqrt(2)) * _erfinv(u))
            return carry

        lax.fori_loop(0, C // GEN, gen_body, 0)

        eps = eps_scr[pl.ds(j * C, C), :]
        d = (loc[...] + scale[...] * eps) - tgt_ref[...]
        q4 = q4_ref[...]
        acc = jnp.zeros((C, 128), jnp.float32)
        for k in range(LG):
            dk = d[:, 128 * k:128 * (k + 1)]
            ek = lax.dot_general(dk, q4, (((1,), (0,)), ((), ())),
                                 preferred_element_type=jnp.float32)
            acc = acc + ek * dk
        vblock = -jnp.sum(acc, axis=1, keepdims=True).reshape(1, C)
        vals[pl.ds(0, 1), pl.ds(j * C, C)] = vblock

    @pl.when(j == NB)
    def _select():
        v = vals[...]
        b = v.view(jnp.int32)
        key = jnp.where(b < 0, b ^ jnp.int32(0x7FFFFFFF), b)
        ukey = key.view(jnp.uint32) ^ jnp.uint32(0x80000000)

        def body(k, t):
            t_try = t | (jnp.uint32(1) << jnp.uint32(31 - k))
            cnt = jnp.sum((ukey >= t_try).astype(jnp.int32))
            return jnp.where(cnt >= NUM_TOPK, t_try, t)

        t = lax.fori_loop(0, 32, body, jnp.uint32(0))
        gt = ukey > t
        eq = ukey == t
        need = NUM_TOPK - jnp.sum(gt.astype(jnp.int32))
        rank = _cumsum_lanes(eq.astype(jnp.float32))
        wsel = gt | (eq & (rank <= need.astype(jnp.float32)))
        w[...] = wsel.astype(jnp.float32)
        kmax = jnp.max(key)
        eqb = key == kmax
        bh[...] = (eqb & (_cumsum_lanes(eqb.astype(jnp.float32)) == 1.0)
                   ).astype(jnp.float32)
        bva[1] = jnp.max(v)
        acc_e[...] = jnp.zeros_like(acc_e)
        acc_e2[...] = jnp.zeros_like(acc_e2)
        acc_b[...] = jnp.zeros_like(acc_b)

    @pl.when(j > NB)
    def _accumulate():
        k = j - NB - 1
        e = eps_scr[pl.ds(k * C, C), :]
        wk = w[pl.ds(0, 1), pl.ds(k * C, C)]
        bhk = bh[pl.ds(0, 1), pl.ds(k * C, C)]
        dn = (((1,), (0,)), ((), ()))
        acc_e[...] += lax.dot_general(wk, e, dn,
                                      preferred_element_type=jnp.float32)
        acc_e2[...] += lax.dot_general(wk, e * e, dn,
                                       preferred_element_type=jnp.float32)
        acc_b[...] += lax.dot_general(bhk, e, dn,
                                      preferred_element_type=jnp.float32)

    @pl.when(j == 2 * NB)
    def _finalize():
        lc = loc[...]
        sc = scale[...]
        inv = jnp.float32(1.0 / NUM_TOPK)
        m_e = acc_e[...] * inv
        m_e2 = acc_e2[...] * inv
        new_means = lc + sc * m_e
        var_eps = (m_e2 - m_e * m_e) * jnp.float32(NUM_TOPK / (NUM_TOPK - 1))
        new_stds = sc * jnp.sqrt(jnp.maximum(var_eps, 0.0))
        best_sample = lc + sc * acc_b[...]
        bvi = bva[1]
        better = bvi > bva[0]
        bva[0] = jnp.where(better, bvi, bva[0])
        besta[...] = jnp.where(better, best_sample, besta[...])
        loc[...] = (jnp.float32(MOMENTUM) * means0_ref[...]
                    + jnp.float32(1.0 - MOMENTUM) * new_means)
        scale[...] = (jnp.float32(MOMENTUM)
                      + jnp.float32(1.0 - MOMENTUM) * new_stds)

    @pl.when((i == NUM_ITERS - 1) & (j == 2 * NB))
    def _emit():
        out_ref[...] = besta[...]


def kernel(initial_solution, target, Q):
    means0 = initial_solution.reshape(1, HA)
    tgt = target.reshape(1, HA)
    q4 = jnp.kron(jnp.eye(4, dtype=jnp.float32), Q)

    base = jax.random.key(42)
    keys = jnp.stack([jax.random.key_data(jax.random.fold_in(base, i))
                      for i in range(NUM_ITERS)]).astype(jnp.uint32)

    out = pl.pallas_call(
        _cem_kernel,
        grid=(NUM_ITERS, 2 * NB + 1),
        in_specs=[
            pl.BlockSpec(memory_space=pltpu.SMEM),
            pl.BlockSpec((1, HA), lambda i, j: (0, 0)),
            pl.BlockSpec((1, HA), lambda i, j: (0, 0)),
            pl.BlockSpec((128, 128), lambda i, j: (0, 0)),
        ],
        out_specs=pl.BlockSpec((1, HA), lambda i, j: (0, 0)),
        out_shape=jax.ShapeDtypeStruct((1, HA), jnp.float32),
        scratch_shapes=[
            pltpu.VMEM((1, POP), jnp.float32),
            pltpu.VMEM((1, POP), jnp.float32),
            pltpu.VMEM((1, POP), jnp.float32),
            pltpu.VMEM((1, HA), jnp.float32),
            pltpu.VMEM((1, HA), jnp.float32),
            pltpu.SMEM((2,), jnp.float32),
            pltpu.VMEM((1, HA), jnp.float32),
            pltpu.VMEM((1, HA), jnp.float32),
            pltpu.VMEM((1, HA), jnp.float32),
            pltpu.VMEM((1, HA), jnp.float32),
            pltpu.VMEM((POP, HA), jnp.float32),
        ],
        compiler_params=pltpu.CompilerParams(
            dimension_semantics=("arbitrary", "arbitrary")),
    )(keys, means0, tgt, q4)
    return out.reshape(H, A)


# speedup vs baseline: 1.4636x; 1.0060x over previous
import math

import jax
import jax.numpy as jnp
from jax import lax
from jax.experimental import pallas as pl
from jax.experimental.pallas import tpu as pltpu

H, A = 100, 32
HA = H * A
POP = 2048
NUM_TOPK = 205
NUM_ITERS = 5
MOMENTUM = 0.1

C = 256
NB = POP // C
LG = HA // 128
GEN = 128
HALF = POP * HA // 2


def _cumsum_lanes(x):
    n = x.shape[1]
    k = 1
    while k < n:
        shifted = jnp.pad(x, ((0, 0), (k, 0)))[:, :n]
        x = x + shifted
        k *= 2
    return x


def _erfinv(x):
    w = -jnp.log((1.0 - x) * (1.0 + x))
    ws = w - 2.5
    p1 = jnp.float32(2.81022636e-08)
    for c in (3.43273939e-07, -3.5233877e-06, -4.39150654e-06,
              0.00021858087, -0.00125372503, -0.00417768164,
              0.246640727, 1.50140941):
        p1 = p1 * ws + jnp.float32(c)
    wl = jnp.sqrt(w) - 3.0
    p2 = jnp.float32(-0.000200214257)
    for c in (0.000100950558, 0.00134934322, -0.00367342844,
              0.00573950773, -0.0076224613, 0.00943887047,
              1.00167406, 2.83297682):
        p2 = p2 * wl + jnp.float32(c)
    return jnp.where(w < 5.0, p1, p2) * x


def _cem_kernel(keys_ref, means0_ref, tgt_ref, q4_ref, out_ref,
                vals, w, bh, loc, scale, bva, besta,
                acc_e, acc_e2, acc_b, eps_scr):
    i = pl.program_id(0)
    j = pl.program_id(1)

    @pl.when((i == 0) & (j == 0))
    def _init():
        loc[...] = means0_ref[...]
        scale[...] = jnp.ones_like(scale)
        bva[0] = -jnp.inf
        besta[...] = jnp.zeros_like(besta)

    @pl.when(j < NB)
    def _generate_and_values():
        k0 = keys_ref[i, 0]
        k1 = keys_ref[i, 1]
        k2 = k0 ^ k1 ^ jnp.uint32(0x1BD11BDA)
        ks = (k0, k1, k2)

        def gen_body(sc, carry):
            r0 = j * C + sc * GEN
            p = ((r0 * HA).astype(jnp.uint32)
                 + lax.broadcasted_iota(jnp.uint32, (GEN, HA), 0)
                 * jnp.uint32(HA)
                 + lax.broadcasted_iota(jnp.uint32, (GEN, HA), 1))
            x0 = jnp.zeros_like(p) + k0
            x1 = p + k1
            rots = ((13, 15, 26, 6), (17, 29, 16, 24))
            for g in range(5):
                for r in rots[g % 2]:
                    x0 = x0 + x1
                    x1 = (x1 << jnp.uint32(r)) | (x1 >> jnp.uint32(32 - r))
                    x1 = x1 ^ x0
                x0 = x0 + ks[(g + 1) % 3]
                x1 = x1 + ks[(g + 2) % 3] + jnp.uint32(g + 1)
            bits = x0 ^ x1
            fl = lax.bitcast_convert_type(
                (bits >> jnp.uint32(9)) | jnp.uint32(0x3F800000), jnp.float32)
            lo = jnp.float32(-0.99999994)
            u = (fl - 1.0) * (jnp.float32(1.0) - lo) + lo
            u = jnp.maximum(lo, u)
            eps_scr[pl.ds(r0, GEN), :] = (
                jnp.float32(math.sqrt(2)) * _erfinv(u))
            return carry

        lax.fori_loop(0, C // GEN, gen_body, 0)

        eps = eps_scr[pl.ds(j * C, C), :]
        d = (loc[...] + scale[...] * eps) - tgt_ref[...]
        q4 = q4_ref[...]
        acc = jnp.zeros((C, 128), jnp.float32)
        for k in range(LG):
            dk = d[:, 128 * k:128 * (k + 1)]
            ek = lax.dot_general(dk, q4, (((1,), (0,)), ((), ())),
                                 preferred_element_type=jnp.float32)
            acc = acc + ek * dk
        vblock = -jnp.sum(acc, axis=1, keepdims=True).reshape(1, C)
        vals[pl.ds(0, 1), pl.ds(j * C, C)] = vblock

    @pl.when(j == NB)
    def _select():
        v = vals[...]
        b = v.view(jnp.int32)
        key = jnp.where(b < 0, b ^ jnp.int32(0x7FFFFFFF), b)
        ukey = key.view(jnp.uint32) ^ jnp.uint32(0x80000000)

        def body(k, t):
            t_try = t | (jnp.uint32(1) << jnp.uint32(31 - k))
            cnt = jnp.sum((ukey >= t_try).astype(jnp.int32))
            return jnp.where(cnt >= NUM_TOPK, t_try, t)

        t = lax.fori_loop(0, 32, body, jnp.uint32(0))
        gt = ukey > t
        eq = ukey == t
        need = NUM_TOPK - jnp.sum(gt.astype(jnp.int32))
        rank = _cumsum_lanes(eq.astype(jnp.float32))
        wsel = gt | (eq & (rank <= need.astype(jnp.float32)))
        w[...] = wsel.astype(jnp.float32)
        kmax = jnp.max(key)
        eqb = key == kmax
        bh[...] = (eqb & (_cumsum_lanes(eqb.astype(jnp.float32)) == 1.0)
                   ).astype(jnp.float32)
        bva[1] = jnp.max(v)
        acc_e[...] = jnp.zeros_like(acc_e)
        acc_e2[...] = jnp.zeros_like(acc_e2)
        acc_b[...] = jnp.zeros_like(acc_b)

    @pl.when(j > NB)
    def _accumulate():
        k = j - NB - 1
        e = eps_scr[pl.ds(k * C, C), :]
        wk = w[pl.ds(0, 1), pl.ds(k * C, C)]
        bhk = bh[pl.ds(0, 1), pl.ds(k * C, C)]
        dn = (((1,), (0,)), ((), ()))
        acc_e[...] += lax.dot_general(wk, e, dn,
                                      preferred_element_type=jnp.float32)
        acc_e2[...] += lax.dot_general(wk, e * e, dn,
                                       preferred_element_type=jnp.float32)
        acc_b[...] += lax.dot_general(bhk, e, dn,
                                      preferred_element_type=jnp.float32)

    @pl.when(j == 2 * NB)
    def _finalize():
        lc = loc[...]
        sc = scale[...]
        inv = jnp.float32(1.0 / NUM_TOPK)
        m_e = acc_e[...] * inv
        m_e2 = acc_e2[...] * inv
        new_means = lc + sc * m_e
        var_eps = (m_e2 - m_e * m_e) * jnp.float32(NUM_TOPK / (NUM_TOPK - 1))
        new_stds = sc * jnp.sqrt(jnp.maximum(var_eps, 0.0))
        best_sample = lc + sc * acc_b[...]
        bvi = bva[1]
        better = bvi > bva[0]
        bva[0] = jnp.where(better, bvi, bva[0])
        besta[...] = jnp.where(better, best_sample, besta[...])
        loc[...] = (jnp.float32(MOMENTUM) * means0_ref[...]
                    + jnp.float32(1.0 - MOMENTUM) * new_means)
        scale[...] = (jnp.float32(MOMENTUM)
                      + jnp.float32(1.0 - MOMENTUM) * new_stds)

    @pl.when((i == NUM_ITERS - 1) & (j == 2 * NB))
    def _emit():
        out_ref[...] = besta[...]


def kernel(initial_solution, target, Q):
    means0 = initial_solution.reshape(1, HA)
    tgt = target.reshape(1, HA)
    q4 = jnp.kron(jnp.eye(4, dtype=jnp.float32), Q)

    base = jax.random.key(42)
    keys = jnp.stack([jax.random.key_data(jax.random.fold_in(base, i))
                      for i in range(NUM_ITERS)]).astype(jnp.uint32)

    out = pl.pallas_call(
        _cem_kernel,
        grid=(NUM_ITERS, 2 * NB + 1),
        in_specs=[
            pl.BlockSpec(memory_space=pltpu.SMEM),
            pl.BlockSpec((1, HA), lambda i, j: (0, 0)),
            pl.BlockSpec((1, HA), lambda i, j: (0, 0)),
            pl.BlockSpec((128, 128), lambda i, j: (0, 0)),
        ],
        out_specs=pl.BlockSpec((1, HA), lambda i, j: (0, 0)),
        out_shape=jax.ShapeDtypeStruct((1, HA), jnp.float32),
        scratch_shapes=[
            pltpu.VMEM((1, POP), jnp.float32),
            pltpu.VMEM((1, POP), jnp.float32),
            pltpu.VMEM((1, POP), jnp.float32),
            pltpu.VMEM((1, HA), jnp.float32),
            pltpu.VMEM((1, HA), jnp.float32),
            pltpu.SMEM((2,), jnp.float32),
            pltpu.VMEM((1, HA), jnp.float32),
            pltpu.VMEM((1, HA), jnp.float32),
            pltpu.VMEM((1, HA), jnp.float32),
            pltpu.VMEM((1, HA), jnp.float32),
            pltpu.VMEM((POP, HA), jnp.float32),
        ],
        compiler_params=pltpu.CompilerParams(
            dimension_semantics=("arbitrary", "arbitrary")),
    )(keys, means0, tgt, q4)
    return out.reshape(H, A)
